# baseline (speedup 1.0000x reference)
import jax
import jax.numpy as jnp
from jax import lax
from jax.experimental import pallas as pl
from jax.experimental.pallas import tpu as pltpu
from jax.experimental.pallas import tpu_sc as plsc

N, C, H, W = 4, 96, 512, 512
HW = H * W
P = N * HW
L = 16
NC, NS = 2, 16
NW = NC * NS
PW = P // NW
CHUNK = 8192
NCHUNK = PW // CHUNK
GROUPS = CHUNK // L
ROW_T = HW // L
ROW_N = C * HW // L
IMGW = HW // PW


def _sc_body(pred_ref, tgt_ref, out_ref,
             t_v, idx0, idx1, rows0, rows1, acc_v, sem0, sem1):
    cid = lax.axis_index("c")
    sid = lax.axis_index("s")
    wid = sid * NC + cid
    n = wid // IMGW

    pltpu.sync_copy(tgt_ref.at[pl.ds(wid * PW, PW)], t_v)

    iota = lax.iota(jnp.int32, L)

    ebase = wid * PW + n * (C - 1) * HW

    def build_idx(c, idx_ref):
        def g_body(j, _):
            for u in range(4):
                o = (j * 4 + u) * L
                tv = t_v[pl.ds(c * CHUNK + o, L)]
                elems = tv * HW + (ebase + c * CHUNK + o) + iota
                idx_ref[pl.ds(o, L)] = elems
            return 0
        lax.fori_loop(0, GROUPS // 4, g_body, 0)

    def extract(rows_ref, acc):
        def e_body(j, acc):
            for u in range(4):
                g = j * 4 + u
                acc = acc + rows_ref[pl.ds(g * L, L)]
            return acc
        return lax.fori_loop(0, GROUPS // 4, e_body, acc)

    idxs = [idx0, idx1]
    rows = [rows0, rows1]
    sems = [sem0, sem1]
    copies = {}

    build_idx(0, idxs[0])
    copies[0] = pltpu.async_copy(pred_ref.at[idxs[0]], rows[0], sems[0])

    acc = jnp.zeros((L,), jnp.float32)
    for c in range(NCHUNK):
        cur = c % 2
        if c + 1 < NCHUNK:
            nxt = (c + 1) % 2
            build_idx(c + 1, idxs[nxt])
            copies[c + 1] = pltpu.async_copy(
                pred_ref.at[idxs[nxt]], rows[nxt], sems[nxt])
        copies[c].wait()
        acc = extract(rows[cur], acc)

    acc_v[...] = acc
    pltpu.sync_copy(acc_v, out_ref.at[pl.ds(wid * L, L)])


@jax.jit
def _nll_sum(table, tgt):
    mesh = plsc.VectorSubcoreMesh(core_axis_name="c", subcore_axis_name="s")
    f = pl.kernel(
        _sc_body,
        out_type=jax.ShapeDtypeStruct((NW * L,), jnp.float32),
        mesh=mesh,
        scratch_types=[
            pltpu.VMEM((PW,), jnp.int32),
            pltpu.VMEM((CHUNK,), jnp.int32),
            pltpu.VMEM((CHUNK,), jnp.int32),
            pltpu.VMEM((CHUNK,), jnp.float32),
            pltpu.VMEM((CHUNK,), jnp.float32),
            pltpu.VMEM((L,), jnp.float32),
            pltpu.SemaphoreType.DMA,
            pltpu.SemaphoreType.DMA,
        ],
    )
    return f(table, tgt)


def kernel(pred, target):
    table = (pred.reshape(N, C, H // 8, 8, W // 128, 128)
             .transpose(0, 1, 2, 4, 3, 5)
             .reshape(-1))
    tgt = (target.astype(jnp.int32)
           .reshape(N, H // 8, 8, W // 128, 128)
           .transpose(0, 1, 3, 2, 4)
           .reshape(-1))
    partials = _nll_sum(table, tgt)
    return -jnp.sum(partials) / P


# speedup vs baseline: 1.0138x; 1.0138x over previous
import jax
import jax.numpy as jnp
from jax import lax
from jax.experimental import pallas as pl
from jax.experimental.pallas import tpu as pltpu
from jax.experimental.pallas import tpu_sc as plsc

N, C, H, W = 4, 96, 512, 512
HW = H * W
P = N * HW
L = 16
NC, NS = 2, 16
NW = NC * NS
PW = P // NW
CHUNK = 4096
NCHUNK = PW // CHUNK
GROUPS = CHUNK // L
ROW_T = HW // L
ROW_N = C * HW // L
IMGW = HW // PW


def _sc_body(pred_ref, tgt_ref, out_ref,
             t_v, idx0, idx1, rows0, rows1, acc_v, sem0, sem1):
    cid = lax.axis_index("c")
    sid = lax.axis_index("s")
    wid = sid * NC + cid
    n = wid // IMGW

    pltpu.sync_copy(tgt_ref.at[pl.ds(wid * PW, PW)], t_v)

    iota = lax.iota(jnp.int32, L)

    ebase = wid * PW + n * (C - 1) * HW

    def build_idx(c, idx_ref):
        def g_body(j, _):
            for u in range(8):
                o = (j * 8 + u) * L
                tv = t_v[pl.ds(c * CHUNK + o, L)]
                elems = tv * HW + (ebase + c * CHUNK + o) + iota
                idx_ref[pl.ds(o, L)] = elems
            return 0
        lax.fori_loop(0, GROUPS // 8, g_body, 0)

    def extract(rows_ref, acc):
        def e_body(j, acc):
            for u in range(8):
                g = j * 8 + u
                acc = acc + rows_ref[pl.ds(g * L, L)]
            return acc
        return lax.fori_loop(0, GROUPS // 8, e_body, acc)

    idxs = [idx0, idx1]
    rows = [rows0, rows1]
    sems = [sem0, sem1]
    copies = {}

    build_idx(0, idxs[0])
    copies[0] = pltpu.async_copy(pred_ref.at[idxs[0]], rows[0], sems[0])

    acc = jnp.zeros((L,), jnp.float32)
    for c in range(NCHUNK):
        cur = c % 2
        if c + 1 < NCHUNK:
            nxt = (c + 1) % 2
            build_idx(c + 1, idxs[nxt])
            copies[c + 1] = pltpu.async_copy(
                pred_ref.at[idxs[nxt]], rows[nxt], sems[nxt])
        copies[c].wait()
        acc = extract(rows[cur], acc)

    acc_v[...] = acc
    pltpu.sync_copy(acc_v, out_ref.at[pl.ds(wid * L, L)])


@jax.jit
def _nll_sum(table, tgt):
    mesh = plsc.VectorSubcoreMesh(core_axis_name="c", subcore_axis_name="s")
    f = pl.kernel(
        _sc_body,
        out_type=jax.ShapeDtypeStruct((NW * L,), jnp.float32),
        mesh=mesh,
        scratch_types=[
            pltpu.VMEM((PW,), jnp.int32),
            pltpu.VMEM((CHUNK,), jnp.int32),
            pltpu.VMEM((CHUNK,), jnp.int32),
            pltpu.VMEM((CHUNK,), jnp.float32),
            pltpu.VMEM((CHUNK,), jnp.float32),
            pltpu.VMEM((L,), jnp.float32),
            pltpu.SemaphoreType.DMA,
            pltpu.SemaphoreType.DMA,
        ],
    )
    return f(table, tgt)


def kernel(pred, target):
    table = (pred.reshape(N, C, H // 8, 8, W // 128, 128)
             .transpose(0, 1, 2, 4, 3, 5)
             .reshape(-1))
    tgt = (target.astype(jnp.int32)
           .reshape(N, H // 8, 8, W // 128, 128)
           .transpose(0, 1, 3, 2, 4)
           .reshape(-1))
    partials = _nll_sum(table, tgt)
    return -jnp.sum(partials) / P


# speedup vs baseline: 1.0317x; 1.0176x over previous
import jax
import jax.numpy as jnp
from jax import lax
from jax.experimental import pallas as pl
from jax.experimental.pallas import tpu as pltpu
from jax.experimental.pallas import tpu_sc as plsc

N, C, H, W = 4, 96, 512, 512
HW = H * W
P = N * HW
L = 16
NC, NS = 2, 16
NW = NC * NS
PW = P // NW
CHUNK = 4096
NCHUNK = PW // CHUNK
GROUPS = CHUNK // L
ROW_T = HW // L
ROW_N = C * HW // L
IMGW = HW // PW


def _sc_body(pred_ref, tgt_ref, out_ref,
             t_v, idx0, idx1, rows0, rows1, acc_v, sem0, sem1, sem2, sem3):
    cid = lax.axis_index("c")
    sid = lax.axis_index("s")
    wid = sid * NC + cid
    n = wid // IMGW

    pltpu.sync_copy(tgt_ref.at[pl.ds(wid * PW, PW)], t_v)

    iota = lax.iota(jnp.int32, L)

    ebase = wid * PW + n * (C - 1) * HW

    def build_idx(c, idx_ref):
        def g_body(j, _):
            for u in range(8):
                o = (j * 8 + u) * L
                tv = t_v[pl.ds(c * CHUNK + o, L)]
                elems = tv * HW + (ebase + c * CHUNK + o) + iota
                idx_ref[pl.ds(o, L)] = elems
            return 0
        lax.fori_loop(0, GROUPS // 8, g_body, 0)

    def extract(rows_ref, acc):
        def e_body(j, acc):
            for u in range(8):
                g = j * 8 + u
                acc = acc + rows_ref[pl.ds(g * L, L)]
            return acc
        return lax.fori_loop(0, GROUPS // 8, e_body, acc)

    idxs = [idx0, idx1]
    rows = [rows0, rows1]
    sems = [(sem0, sem1), (sem2, sem3)]
    copies = {}
    HC = CHUNK // 2

    def fire(c, b):
        sa, sb = sems[b]
        ca = pltpu.async_copy(
            pred_ref.at[idxs[b].at[pl.ds(0, HC)]],
            rows[b].at[pl.ds(0, HC)], sa)
        cb = pltpu.async_copy(
            pred_ref.at[idxs[b].at[pl.ds(HC, HC)]],
            rows[b].at[pl.ds(HC, HC)], sb)
        copies[c] = (ca, cb)

    build_idx(0, idxs[0])
    fire(0, 0)

    acc = jnp.zeros((L,), jnp.float32)
    for c in range(NCHUNK):
        cur = c % 2
        if c + 1 < NCHUNK:
            nxt = (c + 1) % 2
            build_idx(c + 1, idxs[nxt])
            fire(c + 1, nxt)
        copies[c][0].wait()
        copies[c][1].wait()
        acc = extract(rows[cur], acc)

    acc_v[...] = acc
    pltpu.sync_copy(acc_v, out_ref.at[pl.ds(wid * L, L)])


@jax.jit
def _nll_sum(table, tgt):
    mesh = plsc.VectorSubcoreMesh(core_axis_name="c", subcore_axis_name="s")
    f = pl.kernel(
        _sc_body,
        out_type=jax.ShapeDtypeStruct((NW * L,), jnp.float32),
        mesh=mesh,
        scratch_types=[
            pltpu.VMEM((PW,), jnp.int32),
            pltpu.VMEM((CHUNK,), jnp.int32),
            pltpu.VMEM((CHUNK,), jnp.int32),
            pltpu.VMEM((CHUNK,), jnp.float32),
            pltpu.VMEM((CHUNK,), jnp.float32),
            pltpu.VMEM((L,), jnp.float32),
            pltpu.SemaphoreType.DMA,
            pltpu.SemaphoreType.DMA,
            pltpu.SemaphoreType.DMA,
            pltpu.SemaphoreType.DMA,
        ],
    )
    return f(table, tgt)


def kernel(pred, target):
    table = (pred.reshape(N, C, H // 8, 8, W // 128, 128)
             .transpose(0, 1, 2, 4, 3, 5)
             .reshape(-1))
    tgt = (target.astype(jnp.int32)
           .reshape(N, H // 8, 8, W // 128, 128)
           .transpose(0, 1, 3, 2, 4)
           .reshape(-1))
    partials = _nll_sum(table, tgt)
    return -jnp.sum(partials) / P
